# baseline (speedup 1.0000x reference)
import functools

import jax
import jax.numpy as jnp
from jax import lax
from jax.experimental import pallas as pl
from jax.experimental.pallas import tpu as pltpu
from jax.experimental.pallas import tpu_sc as plsc

HIDDEN = 512
B = 16384
NFEAT = 47
VOCAB = 128
EMB = HIDDEN // 4

NW = 32
B_SC = 10240
BT = B_SC // NW
TC_BLK = 256
HEAD_BLOCKS = B_SC // TC_BLK


def _sc_disc_body(tab_hbm, idx_hbm, out_hbm, tab_v, idx_v, out_v):
    wid = lax.axis_index("s") * 2 + lax.axis_index("c")
    base = wid * BT
    pltpu.sync_copy(tab_hbm, tab_v)
    pltpu.sync_copy(idx_hbm.at[pl.ds(base * NFEAT, BT * NFEAT)],
                    idx_v.at[pl.ds(0, BT * NFEAT)])

    def pbody(p, carry):
        pf = p * NFEAT
        iv = [idx_v[pl.ds(pf + 16 * k, 16)] for k in range(3)]
        idxs = [iv[k][j] for k in range(3) for j in range(16)][:NFEAT]
        rows = [idx * 2 for idx in idxs]
        grp = [[], [], [], []]
        for f in range(NFEAT):
            grp[f % 4].append(rows[f])

        po = p * 2
        for u in range(4):
            gacc = []
            for g in range(4):
                a = tab_v[pl.ds(grp[g][0], 2), pl.ds(16 * u, 16)]
                for r in grp[g][1:]:
                    a = jnp.maximum(
                        a, tab_v[pl.ds(r, 2), pl.ds(16 * u, 16)])
                gacc.append(a)
            acc = jnp.maximum(jnp.maximum(gacc[0], gacc[1]),
                              jnp.maximum(gacc[2], gacc[3]))
            out_v[pl.ds(po, 2), pl.ds(16 * u, 16)] = acc
        return carry

    lax.fori_loop(0, BT, pbody, 0)
    pltpu.sync_copy(out_v, out_hbm.at[pl.ds(base * 2, BT * 2)])


def _sc_disc(tab_rows, player_flat):
    mesh = plsc.VectorSubcoreMesh(core_axis_name="c", subcore_axis_name="s")
    k = functools.partial(
        pl.kernel,
        mesh=mesh,
        out_type=jax.ShapeDtypeStruct((B_SC * 2, EMB // 2), jnp.bfloat16),
        scratch_types=[
            pltpu.VMEM((VOCAB * 2, EMB // 2), jnp.bfloat16),
            pltpu.VMEM((BT * NFEAT + 16,), jnp.int32),
            pltpu.VMEM((BT * 2, EMB // 2), jnp.bfloat16),
        ],
    )(_sc_disc_body)
    return k(tab_rows, player_flat)


def _tc_body(p_ref, tab_ref, w_ref, b_ref, o_ref):
    i = pl.program_id(0)
    p = p_ref[...]
    rows = jax.lax.broadcasted_iota(jnp.int32, (TC_BLK, NFEAT), 0) + i * TC_BLK
    p = jnp.where(rows >= B - 2, p // 10, p)
    x = p.astype(jnp.float32) / 99.0
    o_ref[:, EMB:] = jax.lax.dot_general(
        x, w_ref[...], (((1,), (0,)), ((), ())),
        preferred_element_type=jnp.float32) + b_ref[...]

    @pl.when(i >= HEAD_BLOCKS)
    def _():
        tab = tab_ref[...]
        acc = jnp.full((TC_BLK, EMB), -jnp.inf, jnp.float32)
        for f in range(NFEAT):
            col = jax.lax.slice(p, (0, f), (TC_BLK, f + 1))
            oh = (col == jax.lax.broadcasted_iota(
                jnp.int32, (TC_BLK, VOCAB), 1)).astype(jnp.bfloat16)
            emb_f = jax.lax.dot_general(
                oh, tab, (((1,), (0,)), ((), ())),
                preferred_element_type=jnp.float32)
            acc = jnp.maximum(acc, emb_f)
        o_ref[:, :EMB] = acc


def _tc_main(player, tab_bf, wT, b2):
    return pl.pallas_call(
        _tc_body,
        grid=(B // TC_BLK,),
        in_specs=[
            pl.BlockSpec((TC_BLK, NFEAT), lambda i: (i, 0)),
            pl.BlockSpec((VOCAB, EMB), lambda i: (0, 0)),
            pl.BlockSpec((NFEAT, EMB), lambda i: (0, 0)),
            pl.BlockSpec((1, EMB), lambda i: (0, 0)),
        ],
        out_specs=pl.BlockSpec((TC_BLK, 2 * EMB), lambda i: (i, 0)),
        out_shape=jax.ShapeDtypeStruct((B, 2 * EMB), jnp.float32),
    )(player, tab_bf, wT, b2)


def _paste_body(d_ref, buf_ref, o_ref):
    o_ref[:, :EMB] = d_ref[...].astype(jnp.float32)
    o_ref[:, EMB:] = buf_ref[:, EMB:]


def _tc_paste(disc_bf, buf):
    return pl.pallas_call(
        _paste_body,
        grid=(HEAD_BLOCKS,),
        in_specs=[
            pl.BlockSpec((TC_BLK, EMB), lambda i: (i, 0)),
            pl.BlockSpec((TC_BLK, 2 * EMB), lambda i: (i, 0)),
        ],
        out_specs=pl.BlockSpec((TC_BLK, 2 * EMB), lambda i: (i, 0)),
        out_shape=jax.ShapeDtypeStruct((B, 2 * EMB), jnp.float32),
        input_output_aliases={1: 0},
    )(disc_bf, buf)


def kernel(player, embed_table, W_cont, b_cont):
    tab_bf = embed_table.astype(jnp.bfloat16)
    wT = W_cont.T
    b2 = b_cont.reshape(1, EMB)

    disc_bf = _sc_disc(tab_bf.reshape(VOCAB * 2, EMB // 2),
                       player[:B_SC].reshape(B_SC * NFEAT))
    buf = _tc_main(player, tab_bf, wT, b2)
    return _tc_paste(disc_bf.reshape(B_SC, EMB), buf)


# speedup vs baseline: 1.0667x; 1.0667x over previous
import functools

import jax
import jax.numpy as jnp
from jax import lax
from jax.experimental import pallas as pl
from jax.experimental.pallas import tpu as pltpu
from jax.experimental.pallas import tpu_sc as plsc

HIDDEN = 512
B = 16384
NFEAT = 47
VOCAB = 128
EMB = HIDDEN // 4

NW = 32
B_SC = 11264
BT = B_SC // NW
TC_BLK = 256
HEAD_BLOCKS = B_SC // TC_BLK


def _sc_disc_body(tab_hbm, idx_hbm, out_hbm, tab_v, idx_v, out_v):
    wid = lax.axis_index("s") * 2 + lax.axis_index("c")
    base = wid * BT
    pltpu.sync_copy(tab_hbm, tab_v)
    pltpu.sync_copy(idx_hbm.at[pl.ds(base * NFEAT, BT * NFEAT)],
                    idx_v.at[pl.ds(0, BT * NFEAT)])

    def pbody(p, carry):
        pf = p * NFEAT
        iv = [idx_v[pl.ds(pf + 16 * k, 16)] for k in range(3)]
        idxs = [iv[k][j] for k in range(3) for j in range(16)][:NFEAT]
        rows = [idx * 2 for idx in idxs]
        grp = [[], [], [], []]
        for f in range(NFEAT):
            grp[f % 4].append(rows[f])

        po = p * 2
        for u in range(4):
            gacc = []
            for g in range(4):
                a = tab_v[pl.ds(grp[g][0], 2), pl.ds(16 * u, 16)]
                for r in grp[g][1:]:
                    a = jnp.maximum(
                        a, tab_v[pl.ds(r, 2), pl.ds(16 * u, 16)])
                gacc.append(a)
            acc = jnp.maximum(jnp.maximum(gacc[0], gacc[1]),
                              jnp.maximum(gacc[2], gacc[3]))
            out_v[pl.ds(po, 2), pl.ds(16 * u, 16)] = acc
        return carry

    lax.fori_loop(0, BT, pbody, 0)
    pltpu.sync_copy(out_v, out_hbm.at[pl.ds(base * 2, BT * 2)])


def _sc_disc(tab_rows, player_flat):
    mesh = plsc.VectorSubcoreMesh(core_axis_name="c", subcore_axis_name="s")
    k = functools.partial(
        pl.kernel,
        mesh=mesh,
        out_type=jax.ShapeDtypeStruct((B_SC * 2, EMB // 2), jnp.bfloat16),
        scratch_types=[
            pltpu.VMEM((VOCAB * 2, EMB // 2), jnp.bfloat16),
            pltpu.VMEM((BT * NFEAT + 16,), jnp.int32),
            pltpu.VMEM((BT * 2, EMB // 2), jnp.bfloat16),
        ],
    )(_sc_disc_body)
    return k(tab_rows, player_flat)


def _adjust(p, i, base):
    rows = jax.lax.broadcasted_iota(jnp.int32, (TC_BLK, NFEAT), 0) \
        + i * TC_BLK + base
    return jnp.where(rows >= B - 2, p // 10, p)


def _cont_body(p_ref, w_ref, b_ref, o_ref):
    p = _adjust(p_ref[...], pl.program_id(0), 0)
    x = p.astype(jnp.float32) / 99.0
    o_ref[...] = jax.lax.dot_general(
        x, w_ref[...], (((1,), (0,)), ((), ())),
        preferred_element_type=jnp.float32) + b_ref[...]


def _tc_cont(player, wT, b2):
    return pl.pallas_call(
        _cont_body,
        grid=(B // TC_BLK,),
        in_specs=[
            pl.BlockSpec((TC_BLK, NFEAT), lambda i: (i, 0)),
            pl.BlockSpec((NFEAT, EMB), lambda i: (0, 0)),
            pl.BlockSpec((1, EMB), lambda i: (0, 0)),
        ],
        out_specs=pl.BlockSpec((TC_BLK, EMB), lambda i: (i, 0)),
        out_shape=jax.ShapeDtypeStruct((B, EMB), jnp.float32),
    )(player, wT, b2)


def _tail_body(p_ref, tab_ref, o_ref):
    p = _adjust(p_ref[...], pl.program_id(0), B_SC)
    tab = tab_ref[...]
    acc = jnp.full((TC_BLK, EMB), -jnp.inf, jnp.float32)
    for f in range(NFEAT):
        col = jax.lax.slice(p, (0, f), (TC_BLK, f + 1))
        oh = (col == jax.lax.broadcasted_iota(
            jnp.int32, (TC_BLK, VOCAB), 1)).astype(jnp.bfloat16)
        emb_f = jax.lax.dot_general(
            oh, tab, (((1,), (0,)), ((), ())),
            preferred_element_type=jnp.float32)
        acc = jnp.maximum(acc, emb_f)
    o_ref[...] = acc


def _tc_tail(player, tab_bf):
    n = B - B_SC
    return pl.pallas_call(
        _tail_body,
        grid=(n // TC_BLK,),
        in_specs=[
            pl.BlockSpec((TC_BLK, NFEAT), lambda i: (i + HEAD_BLOCKS, 0)),
            pl.BlockSpec((VOCAB, EMB), lambda i: (0, 0)),
        ],
        out_specs=pl.BlockSpec((TC_BLK, EMB), lambda i: (i, 0)),
        out_shape=jax.ShapeDtypeStruct((n, EMB), jnp.float32),
    )(player, tab_bf)


def kernel(player, embed_table, W_cont, b_cont):
    tab_bf = embed_table.astype(jnp.bfloat16)
    wT = W_cont.T
    b2 = b_cont.reshape(1, EMB)

    disc_head = _sc_disc(tab_bf.reshape(VOCAB * 2, EMB // 2),
                         player[:B_SC].reshape(B_SC * NFEAT))
    cont = _tc_cont(player, wT, b2)
    disc_tail = _tc_tail(player, tab_bf)
    disc = jnp.concatenate(
        [disc_head.reshape(B_SC, EMB).astype(jnp.float32), disc_tail], axis=0)
    return jnp.concatenate([disc, cont], axis=1)
